# baseline (speedup 1.0000x reference)
import functools
import math

import jax
import jax.numpy as jnp
from jax import lax
from jax.experimental import pallas as pl
from jax.experimental.pallas import tpu as pltpu
from jax.experimental.pallas import tpu_sc as plsc

_N_TOKENS = 32768
_N_FEAT = 64
_N_GROUPS = 16
_N_EXPERTS = 48
_SHARP = 16.0
_L = 16

_SQRT2 = math.sqrt(2.0)

_ALPHA = (0.00022905065861350646, 0.0034082910107109506, 0.050955695062380861,
          0.18520832239976145, 1.128379143519084)
_BETA = (-1.1791602954361697e-7, 0.000023547966471313185, 0.0010179625278914885,
         0.014070470171167667, 0.11098505178285362, 0.49746925110067538, 1.0)


def _ratio(x):
    y = x / _SQRT2
    y = jnp.minimum(jnp.maximum(y, jnp.float32(-4.0)), jnp.float32(4.0))
    y2 = y * y
    a = jnp.full_like(y2, _ALPHA[0])
    for c in _ALPHA[1:]:
        a = a * y2 + jnp.float32(c)
    b = jnp.full_like(y2, _BETA[0])
    for c in _BETA[1:]:
        b = b * y2 + jnp.float32(c)
    e = (y * a) / b
    r = 0.5 * (1.0 + e)
    return jnp.minimum(jnp.maximum(r, jnp.float32(0.0)), jnp.float32(1.0))


def _sqrt(v):
    vs = jnp.maximum(v, jnp.float32(1e-35))
    vi = lax.bitcast_convert_type(vs, jnp.int32)
    yi = jnp.int32(0x5F3759DF) - lax.shift_right_arithmetic(vi, 1)
    y = lax.bitcast_convert_type(yi, jnp.float32)
    half_v = 0.5 * vs
    for _ in range(3):
        y = y * (1.5 - half_v * y * y)
    return v * y


def _sc_router(x_hbm, out_hbm, x_v, o_v):
    info = plsc.get_sparse_core_info()
    nc = info.num_cores
    wid = lax.axis_index("s") * nc + lax.axis_index("c")
    tpw = _N_TOKENS // (nc * info.num_subcores)
    base = wid * tpw

    pltpu.sync_copy(x_hbm.at[pl.ds(base * _N_FEAT, tpw * _N_FEAT)], x_v)

    lane = lax.iota(jnp.int32, _L)
    zeros = jnp.zeros((_L,), jnp.float32)
    neginf = jnp.full((_L,), -3e38, jnp.float32)
    izeros = jnp.zeros((_L,), jnp.int32)

    def microbatch(mb, _):
        row = mb * _L + lane
        rowf = row * _N_FEAT

        def group(g, carry):
            v1, v2, i1, i2 = carry
            r = []
            for j in range(4):
                xf = plsc.load_gather(x_v, [rowf + (4 * g + j)])
                r.append(_ratio(xf))
            mean = (r[0] + r[1] + r[2] + r[3]) * 0.25
            d = [(ri - mean) * (ri - mean) for ri in r]
            var = (d[0] + d[1] + d[2] + d[3]) * 0.25
            std = _sqrt(var)
            vmax = jnp.maximum(jnp.maximum(r[0], r[1]), jnp.maximum(r[2], r[3]))
            vmin = jnp.minimum(jnp.minimum(r[0], r[1]), jnp.minimum(r[2], r[3]))
            vrange = vmax - vmin
            peak = vmax - mean

            def frac(pred):
                t0 = jnp.where(pred(r[0]), 1.0, 0.0)
                t1 = jnp.where(pred(r[1]), 1.0, 0.0)
                t2 = jnp.where(pred(r[2]), 1.0, 0.0)
                t3 = jnp.where(pred(r[3]), 1.0, 0.0)
                return (t0 + t1 + t2 + t3) * jnp.float32(0.25)

            zf = frac(lambda ri: ri <= 0.1)
            lf = frac(lambda ri: ri <= 0.25)
            mf = frac(lambda ri: (ri >= 0.3) & (ri <= 0.7))
            hf = frac(lambda ri: ri >= 0.75)
            sf = frac(lambda ri: ri >= 0.9)

            dl = mean - 0.18
            dm = mean - 0.5
            dh = mean - 0.78
            act = (0.9 * mean + 0.7 * hf + 0.4 * sf + 0.5 * peak + 0.35 * std
                   + 0.2 * vrange - 0.8 * zf)
            low = (1.1 * zf + 0.7 * lf - _SHARP * (dl * dl) - 1.1 * std
                   - 0.4 * peak - 0.3 * hf)
            mid = (0.8 * mf - _SHARP * (dm * dm) - 0.5 * std
                   - 0.2 * vrange - 0.2 * peak)
            high = (1.2 * hf + 0.9 * sf + 0.7 * peak + 0.5 * std + 0.3 * vrange
                    - _SHARP * (dh * dh) - 0.3 * zf)

            for j, e in enumerate((act + low, act + mid, act + high)):
                eidx = izeros + (3 * g + j)
                gt1 = e > v1
                gt2 = e > v2
                v2 = jnp.where(gt1, v1, jnp.where(gt2, e, v2))
                i2 = jnp.where(gt1, i1, jnp.where(gt2, eidx, i2))
                v1 = jnp.where(gt1, e, v1)
                i1 = jnp.where(gt1, eidx, i1)
            return v1, v2, i1, i2

        v1, v2, i1, i2 = lax.fori_loop(
            0, _N_GROUPS, group, (neginf, neginf, izeros, izeros), unroll=4)

        s = jnp.exp(v2 - v1)
        denom = 1.0 + s
        w1 = 1.0 / denom
        w2 = s / denom

        obase = mb * (_L * _N_EXPERTS)
        for t in range(_L * _N_EXPERTS // _L):
            o_v[pl.ds(obase + t * _L, _L)] = zeros
        rowe = row * _N_EXPERTS
        plsc.store_scatter(o_v, [rowe + i1], w1)
        plsc.store_scatter(o_v, [rowe + i2], w2)
        return ()

    lax.fori_loop(0, tpw // _L, microbatch, ())

    pltpu.sync_copy(o_v, out_hbm.at[pl.ds(base * _N_EXPERTS, tpw * _N_EXPERTS)])


@jax.jit
def kernel(rule_features):
    n = rule_features.shape[0]
    info = plsc.get_sparse_core_info()
    tpw = n // (info.num_cores * info.num_subcores)
    mesh = plsc.VectorSubcoreMesh(core_axis_name="c", subcore_axis_name="s")
    f = functools.partial(
        pl.kernel,
        out_type=jax.ShapeDtypeStruct((n * _N_EXPERTS,), jnp.float32),
        scratch_types=[
            pltpu.VMEM((tpw * _N_FEAT,), jnp.float32),
            pltpu.VMEM((tpw * _N_EXPERTS,), jnp.float32),
        ],
        mesh=mesh,
        compiler_params=pltpu.CompilerParams(
            use_tc_tiling_on_sc=False, needs_layout_passes=False),
    )(_sc_router)
    flat = f(rule_features.reshape(-1))
    return flat.reshape(n, _N_EXPERTS)


# speedup vs baseline: 1.2531x; 1.2531x over previous
import functools
import math

import jax
import jax.numpy as jnp
from jax import lax
from jax.experimental import pallas as pl
from jax.experimental.pallas import tpu as pltpu
from jax.experimental.pallas import tpu_sc as plsc

_N_TOKENS = 32768
_N_FEAT = 64
_N_GROUPS = 16
_N_EXPERTS = 48
_SHARP = 16.0
_L = 16

_SQRT2 = math.sqrt(2.0)

_ALPHA = (0.00022905065861350646, 0.0034082910107109506, 0.050955695062380861,
          0.18520832239976145, 1.128379143519084)
_BETA = (-1.1791602954361697e-7, 0.000023547966471313185, 0.0010179625278914885,
         0.014070470171167667, 0.11098505178285362, 0.49746925110067538, 1.0)


def _ratio(x):
    y = x / _SQRT2
    y = jnp.minimum(jnp.maximum(y, jnp.float32(-4.0)), jnp.float32(4.0))
    y2 = y * y
    a = jnp.full_like(y2, _ALPHA[0])
    for c in _ALPHA[1:]:
        a = a * y2 + jnp.float32(c)
    b = jnp.full_like(y2, _BETA[0])
    for c in _BETA[1:]:
        b = b * y2 + jnp.float32(c)
    e = (y * a) / b
    r = 0.5 * (1.0 + e)
    return jnp.minimum(jnp.maximum(r, jnp.float32(0.0)), jnp.float32(1.0))


def _sqrt(v):
    vs = jnp.maximum(v, jnp.float32(1e-35))
    vi = lax.bitcast_convert_type(vs, jnp.int32)
    yi = jnp.int32(0x5F3759DF) - lax.shift_right_arithmetic(vi, 1)
    y = lax.bitcast_convert_type(yi, jnp.float32)
    half_v = 0.5 * vs
    for _ in range(3):
        y = y * (1.5 - half_v * y * y)
    return v * y


def _sc_router(x_hbm, out_hbm, x_v, o_v):
    info = plsc.get_sparse_core_info()
    nc = info.num_cores
    wid = lax.axis_index("s") * nc + lax.axis_index("c")
    tpw = _N_TOKENS // (nc * info.num_subcores)
    base = wid * tpw

    pltpu.sync_copy(x_hbm.at[:, pl.ds(base, tpw)], x_v)

    lane = lax.iota(jnp.int32, _L)
    zeros = jnp.zeros((_L,), jnp.float32)
    neginf = jnp.full((_L,), -3e38, jnp.float32)
    izeros = jnp.zeros((_L,), jnp.int32)

    def microbatch(mb, _):
        row = mb * _L + lane
        tok = mb * _L

        def group(g, carry):
            v1, v2, i1, i2 = carry
            r = []
            for j in range(4):
                xf = x_v[4 * g + j, pl.ds(tok, _L)]
                r.append(_ratio(xf))
            mean = (r[0] + r[1] + r[2] + r[3]) * 0.25
            d = [(ri - mean) * (ri - mean) for ri in r]
            var = (d[0] + d[1] + d[2] + d[3]) * 0.25
            std = _sqrt(var)
            vmax = jnp.maximum(jnp.maximum(r[0], r[1]), jnp.maximum(r[2], r[3]))
            vmin = jnp.minimum(jnp.minimum(r[0], r[1]), jnp.minimum(r[2], r[3]))
            vrange = vmax - vmin
            peak = vmax - mean

            def frac(pred):
                t0 = jnp.where(pred(r[0]), 1.0, 0.0)
                t1 = jnp.where(pred(r[1]), 1.0, 0.0)
                t2 = jnp.where(pred(r[2]), 1.0, 0.0)
                t3 = jnp.where(pred(r[3]), 1.0, 0.0)
                return (t0 + t1 + t2 + t3) * jnp.float32(0.25)

            zf = frac(lambda ri: ri <= 0.1)
            lf = frac(lambda ri: ri <= 0.25)
            mf = frac(lambda ri: (ri >= 0.3) & (ri <= 0.7))
            hf = frac(lambda ri: ri >= 0.75)
            sf = frac(lambda ri: ri >= 0.9)

            dl = mean - 0.18
            dm = mean - 0.5
            dh = mean - 0.78
            act = (0.9 * mean + 0.7 * hf + 0.4 * sf + 0.5 * peak + 0.35 * std
                   + 0.2 * vrange - 0.8 * zf)
            low = (1.1 * zf + 0.7 * lf - _SHARP * (dl * dl) - 1.1 * std
                   - 0.4 * peak - 0.3 * hf)
            mid = (0.8 * mf - _SHARP * (dm * dm) - 0.5 * std
                   - 0.2 * vrange - 0.2 * peak)
            high = (1.2 * hf + 0.9 * sf + 0.7 * peak + 0.5 * std + 0.3 * vrange
                    - _SHARP * (dh * dh) - 0.3 * zf)

            for j, e in enumerate((act + low, act + mid, act + high)):
                eidx = izeros + (3 * g + j)
                gt1 = e > v1
                gt2 = e > v2
                v2 = jnp.where(gt1, v1, jnp.where(gt2, e, v2))
                i2 = jnp.where(gt1, i1, jnp.where(gt2, eidx, i2))
                v1 = jnp.where(gt1, e, v1)
                i1 = jnp.where(gt1, eidx, i1)
            return v1, v2, i1, i2

        v1, v2, i1, i2 = lax.fori_loop(
            0, _N_GROUPS, group, (neginf, neginf, izeros, izeros))

        s = jnp.exp(v2 - v1)
        denom = 1.0 + s
        w1 = 1.0 / denom
        w2 = s / denom

        obase = mb * (_L * _N_EXPERTS)
        for t in range(_L * _N_EXPERTS // _L):
            o_v[pl.ds(obase + t * _L, _L)] = zeros
        rowe = row * _N_EXPERTS
        plsc.store_scatter(o_v, [rowe + i1], w1)
        plsc.store_scatter(o_v, [rowe + i2], w2)
        return ()

    lax.fori_loop(0, tpw // _L, microbatch, ())

    pltpu.sync_copy(o_v, out_hbm.at[pl.ds(base * _N_EXPERTS, tpw * _N_EXPERTS)])


@jax.jit
def kernel(rule_features):
    n = rule_features.shape[0]
    info = plsc.get_sparse_core_info()
    tpw = n // (info.num_cores * info.num_subcores)
    mesh = plsc.VectorSubcoreMesh(core_axis_name="c", subcore_axis_name="s")
    f = functools.partial(
        pl.kernel,
        out_type=jax.ShapeDtypeStruct((n * _N_EXPERTS,), jnp.float32),
        scratch_types=[
            pltpu.VMEM((_N_FEAT, tpw), jnp.float32),
            pltpu.VMEM((tpw * _N_EXPERTS,), jnp.float32),
        ],
        mesh=mesh,
        compiler_params=pltpu.CompilerParams(
            use_tc_tiling_on_sc=False, needs_layout_passes=False),
    )(_sc_router)
    flat = f(rule_features.T)
    return flat.reshape(n, _N_EXPERTS)


# speedup vs baseline: 1.3708x; 1.0939x over previous
import functools
import math

import jax
import jax.numpy as jnp
from jax import lax
from jax.experimental import pallas as pl
from jax.experimental.pallas import tpu as pltpu
from jax.experimental.pallas import tpu_sc as plsc

_N_TOKENS = 32768
_N_FEAT = 64
_N_GROUPS = 16
_N_EXPERTS = 48
_SHARP = 16.0
_L = 16

_SQRT2 = math.sqrt(2.0)

_ALPHA = (0.00022905065861350646, 0.0034082910107109506, 0.050955695062380861,
          0.18520832239976145, 1.128379143519084)
_BETA = (-1.1791602954361697e-7, 0.000023547966471313185, 0.0010179625278914885,
         0.014070470171167667, 0.11098505178285362, 0.49746925110067538, 1.0)


def _ratio(x):
    y = x / _SQRT2
    y = jnp.minimum(jnp.maximum(y, jnp.float32(-4.0)), jnp.float32(4.0))
    y2 = y * y
    a = jnp.full_like(y2, _ALPHA[0])
    for c in _ALPHA[1:]:
        a = a * y2 + jnp.float32(c)
    b = jnp.full_like(y2, _BETA[0])
    for c in _BETA[1:]:
        b = b * y2 + jnp.float32(c)
    e = (y * a) / b
    r = 0.5 * (1.0 + e)
    return jnp.minimum(jnp.maximum(r, jnp.float32(0.0)), jnp.float32(1.0))


def _sqrt(v):
    vs = jnp.maximum(v, jnp.float32(1e-35))
    vi = lax.bitcast_convert_type(vs, jnp.int32)
    yi = jnp.int32(0x5F3759DF) - lax.shift_right_arithmetic(vi, 1)
    y = lax.bitcast_convert_type(yi, jnp.float32)
    half_v = 0.5 * vs
    for _ in range(3):
        y = y * (1.5 - half_v * y * y)
    return v * y


def _sc_router(x_hbm, out_hbm, x_v, o_v):
    info = plsc.get_sparse_core_info()
    nc = info.num_cores
    wid = lax.axis_index("s") * nc + lax.axis_index("c")
    tpw = _N_TOKENS // (nc * info.num_subcores)
    base = wid * tpw

    pltpu.sync_copy(x_hbm.at[:, pl.ds(base, tpw)], x_v)

    lane = lax.iota(jnp.int32, _L)
    zeros = jnp.zeros((_L,), jnp.float32)
    neginf = jnp.full((_L,), -3e38, jnp.float32)
    izeros = jnp.zeros((_L,), jnp.int32)

    def microbatch(mb, _):
        row = mb * _L + lane
        tok = mb * _L

        def group(g, carry):
            v1, v2, i1, i2 = carry
            r = []
            for j in range(4):
                xf = x_v[4 * g + j, pl.ds(tok, _L)]
                r.append(_ratio(xf))
            mean = (r[0] + r[1] + r[2] + r[3]) * 0.25
            d = [(ri - mean) * (ri - mean) for ri in r]
            var = (d[0] + d[1] + d[2] + d[3]) * 0.25
            std = _sqrt(var)
            vmax = jnp.maximum(jnp.maximum(r[0], r[1]), jnp.maximum(r[2], r[3]))
            vmin = jnp.minimum(jnp.minimum(r[0], r[1]), jnp.minimum(r[2], r[3]))
            vrange = vmax - vmin
            peak = vmax - mean

            def frac(pred):
                t0 = jnp.where(pred(r[0]), 0.25, 0.0)
                t1 = jnp.where(pred(r[1]), 0.25, 0.0)
                t2 = jnp.where(pred(r[2]), 0.25, 0.0)
                t3 = jnp.where(pred(r[3]), 0.25, 0.0)
                return (t0 + t1) + (t2 + t3)

            zf = frac(lambda ri: ri <= 0.1)
            lf = frac(lambda ri: ri <= 0.25)
            mf = frac(lambda ri: jnp.abs(ri - 0.5) <= 0.2)
            hf = frac(lambda ri: ri >= 0.75)
            sf = frac(lambda ri: ri >= 0.9)

            dl = mean - 0.18
            dm = mean - 0.5
            dh = mean - 0.78
            act = (0.9 * mean + 0.7 * hf + 0.4 * sf + 0.5 * peak + 0.35 * std
                   + 0.2 * vrange - 0.8 * zf)
            low = (1.1 * zf + 0.7 * lf - _SHARP * (dl * dl) - 1.1 * std
                   - 0.4 * peak - 0.3 * hf)
            mid = (0.8 * mf - _SHARP * (dm * dm) - 0.5 * std
                   - 0.2 * vrange - 0.2 * peak)
            high = (1.2 * hf + 0.9 * sf + 0.7 * peak + 0.5 * std + 0.3 * vrange
                    - _SHARP * (dh * dh) - 0.3 * zf)

            for j, e in enumerate((act + low, act + mid, act + high)):
                eidx = izeros + (3 * g + j)
                gt1 = e > v1
                gt2 = e > v2
                v2 = jnp.where(gt1, v1, jnp.where(gt2, e, v2))
                i2 = jnp.where(gt1, i1, jnp.where(gt2, eidx, i2))
                v1 = jnp.where(gt1, e, v1)
                i1 = jnp.where(gt1, eidx, i1)
            return v1, v2, i1, i2

        v1, v2, i1, i2 = lax.fori_loop(
            0, _N_GROUPS, group, (neginf, neginf, izeros, izeros))

        s = jnp.exp(v2 - v1)
        denom = 1.0 + s
        w1 = 1.0 / denom
        w2 = s / denom

        obase = mb * (_L * _N_EXPERTS)
        for t in range(_L * _N_EXPERTS // _L):
            o_v[pl.ds(obase + t * _L, _L)] = zeros
        rowe = row * _N_EXPERTS
        plsc.store_scatter(o_v, [rowe + i1], w1)
        plsc.store_scatter(o_v, [rowe + i2], w2)
        return ()

    lax.fori_loop(0, tpw // _L, microbatch, ())

    pltpu.sync_copy(o_v, out_hbm.at[pl.ds(base * _N_EXPERTS, tpw * _N_EXPERTS)])


@jax.jit
def kernel(rule_features):
    n = rule_features.shape[0]
    info = plsc.get_sparse_core_info()
    tpw = n // (info.num_cores * info.num_subcores)
    mesh = plsc.VectorSubcoreMesh(core_axis_name="c", subcore_axis_name="s")
    f = functools.partial(
        pl.kernel,
        out_type=jax.ShapeDtypeStruct((n * _N_EXPERTS,), jnp.float32),
        scratch_types=[
            pltpu.VMEM((_N_FEAT, tpw), jnp.float32),
            pltpu.VMEM((tpw * _N_EXPERTS,), jnp.float32),
        ],
        mesh=mesh,
        compiler_params=pltpu.CompilerParams(
            use_tc_tiling_on_sc=False, needs_layout_passes=False),
    )(_sc_router)
    flat = f(rule_features.T)
    return flat.reshape(n, _N_EXPERTS)
